# Initial kernel scaffold; baseline (speedup 1.0000x reference)
import jax
import jax.numpy as jnp
from jax.experimental import pallas as pl


def kernel(hidden_states, gate_w, gate_b, w_gate_up, w_down, shared_gate_up, shared_down):
    raise NotImplementedError("write your pallas kernel here")


# speedup vs baseline: 1.0964x; 1.0964x over previous
import functools

import jax
import jax.numpy as jnp
from jax import lax
from jax.experimental import pallas as pl
from jax.experimental.pallas import tpu as pltpu

_T = 2048
_H = 1024
_E = 16
_K = 2
_NG = 4
_I = 512
_IS = 1024
_SCALE = 2.5

_B = 128
_NB = (_K * _T) // _B + _E
_P = _NB * _B


def _gate_body(x_ref, w_ref, b_ref, out_ref):
    logits = lax.dot_general(w_ref[...], x_ref[...], (((1,), (1,)), ((), ())),
                             preferred_element_type=jnp.float32)
    out_ref[...] = jax.nn.sigmoid(logits + b_ref[...])


def _gate_scores_t(x, gate_w, gate_b):
    return pl.pallas_call(
        _gate_body,
        out_shape=jax.ShapeDtypeStruct((_E, _T), jnp.float32),
    )(x, gate_w, gate_b.reshape(_E, 1))


def _shared_body(x_ref, wgu_ref, wd_ref, out_ref):
    gu = lax.dot_general(x_ref[...], wgu_ref[...], (((1,), (1,)), ((), ())),
                         preferred_element_type=jnp.float32)
    g = gu[:, :_IS]
    u = gu[:, _IS:]
    h = g * jax.nn.sigmoid(g) * u
    out_ref[...] = lax.dot_general(h, wd_ref[...], (((1,), (1,)), ((), ())),
                                   preferred_element_type=jnp.float32)


def _shared_mlp(x, shared_gate_up, shared_down):
    tb = 256
    return pl.pallas_call(
        _shared_body,
        grid=(_T // tb,),
        in_specs=[
            pl.BlockSpec((tb, _H), lambda i: (i, 0)),
            pl.BlockSpec((2 * _IS, _H), lambda i: (0, 0)),
            pl.BlockSpec((_H, _IS), lambda i: (0, 0)),
        ],
        out_specs=pl.BlockSpec((tb, _H), lambda i: (i, 0)),
        out_shape=jax.ShapeDtypeStruct((_T, _H), jnp.float32),
    )(x, shared_gate_up, shared_down)


def _expert_body(be_ref, bv_ref, xs_ref, wgu_ref, wd_ref, y_ref):
    @pl.when(bv_ref[pl.program_id(0)] != 0)
    def _():
        gu = lax.dot_general(xs_ref[...], wgu_ref[0], (((1,), (1,)), ((), ())),
                             preferred_element_type=jnp.float32)
        g = gu[:, :_I]
        u = gu[:, _I:]
        h = g * jax.nn.sigmoid(g) * u
        y_ref[...] = lax.dot_general(h, wd_ref[0], (((1,), (1,)), ((), ())),
                                     preferred_element_type=jnp.float32)


def _expert_mm(x_sorted, block_expert, block_valid, w_gate_up, w_down):
    grid_spec = pltpu.PrefetchScalarGridSpec(
        num_scalar_prefetch=2,
        grid=(_NB,),
        in_specs=[
            pl.BlockSpec((_B, _H), lambda i, be, bv: (i, 0)),
            pl.BlockSpec((1, 2 * _I, _H), lambda i, be, bv: (be[i], 0, 0)),
            pl.BlockSpec((1, _H, _I), lambda i, be, bv: (be[i], 0, 0)),
        ],
        out_specs=pl.BlockSpec((_B, _H), lambda i, be, bv: (i, 0)),
    )
    return pl.pallas_call(
        _expert_body,
        grid_spec=grid_spec,
        out_shape=jax.ShapeDtypeStruct((_P, _H), jnp.float32),
    )(block_expert, block_valid, x_sorted, w_gate_up, w_down)


def _route_dispatch(scores_t, gate_b, x):
    scores = scores_t.T
    sfc = scores + gate_b[None, :]
    grouped = sfc.reshape(_T, _NG, _E // _NG)
    top2 = lax.top_k(grouped, 2)[0]
    group_scores = top2.sum(-1)
    group_idx = lax.top_k(group_scores, 2)[1]
    gmask = jnp.zeros((_T, _NG), jnp.float32).at[
        jnp.arange(_T)[:, None], group_idx].set(1.0)
    smask = jnp.repeat(gmask, _E // _NG, axis=1)
    masked = jnp.where(smask > 0, sfc, -jnp.inf)
    topk_ids = lax.top_k(masked, _K)[1]
    topk_w = jnp.take_along_axis(scores, topk_ids, axis=1)
    topk_w = topk_w / (topk_w.sum(-1, keepdims=True) + 1e-20) * _SCALE

    e_flat = topk_ids.reshape(-1)
    order = jnp.argsort(e_flat, stable=True)
    e_sorted = e_flat[order]
    counts = jnp.bincount(e_flat, length=_E)
    padded = (counts + _B - 1) // _B * _B
    starts = jnp.concatenate([jnp.zeros((1,), jnp.int32),
                              jnp.cumsum(padded)[:-1].astype(jnp.int32)])
    sstarts = jnp.concatenate([jnp.zeros((1,), jnp.int32),
                               jnp.cumsum(counts)[:-1].astype(jnp.int32)])
    pos_sorted = (jnp.arange(_K * _T, dtype=jnp.int32)
                  - sstarts[e_sorted] + starts[e_sorted])
    slot = jnp.zeros((_K * _T,), jnp.int32).at[order].set(pos_sorted)
    x_sorted = jnp.zeros((_P, _H), x.dtype).at[pos_sorted].set(x[order // _K])

    nb_used = (starts[-1] + padded[-1]) // _B
    be = jnp.clip(jnp.searchsorted(starts, jnp.arange(_NB) * _B,
                                   side='right') - 1, 0, _E - 1).astype(jnp.int32)
    bv = (jnp.arange(_NB) < nb_used).astype(jnp.int32)
    return topk_w, slot.reshape(_T, _K), x_sorted, be, bv


def kernel(hidden_states, gate_w, gate_b, w_gate_up, w_down,
           shared_gate_up, shared_down):
    x = hidden_states
    scores_t = _gate_scores_t(x, gate_w, gate_b)
    shared_out = _shared_mlp(x, shared_gate_up, shared_down)
    topk_w, slot, x_sorted, be, bv = _route_dispatch(scores_t, gate_b, x)
    y_sorted = _expert_mm(x_sorted, be, bv, w_gate_up, w_down)
    routed = (topk_w[..., None] * y_sorted[slot]).sum(1)
    return routed + shared_out


# speedup vs baseline: 1.8803x; 1.7149x over previous
import functools

import jax
import jax.numpy as jnp
from jax import lax
from jax.experimental import pallas as pl
from jax.experimental.pallas import tpu as pltpu
from jax.experimental.pallas import tpu_sc as plsc

_T = 2048
_H = 1024
_E = 16
_K = 2
_NG = 4
_I = 512
_IS = 1024
_SCALE = 2.5

_B = 128
_NB = (_K * _T) // _B + _E
_P = _NB * _B

_NEG = -1e30

_DTILES = 16
_DTPT = _T // _DTILES
_DNCH = _DTPT // 16
_CTILES = 32
_CTPT = _T // _CTILES


def _iota16():
    return lax.broadcasted_iota(jnp.int32, (16,), 0)


def _take16(vec, idx):
    return lax.gather(
        vec, idx[:, None],
        lax.GatherDimensionNumbers(offset_dims=(), collapsed_slice_dims=(0,),
                                   start_index_map=(0,)),
        (1,), mode=lax.GatherScatterMode.PROMISE_IN_BOUNDS)


def _cumsum16(x):
    it = _iota16()
    for s in (1, 2, 4, 8):
        y = _take16(x, jnp.maximum(it - s, 0))
        x = x + jnp.where(it >= s, y, 0)
    return x


def _sum16v(x):
    return _take16(_cumsum16(x), jnp.full((16,), 15, jnp.int32))


def _gate_body(x_ref, w_ref, b_ref, out_ref):
    logits = lax.dot_general(w_ref[...], x_ref[...], (((1,), (1,)), ((), ())),
                             preferred_element_type=jnp.float32)
    out_ref[...] = jax.nn.sigmoid(logits + b_ref[...])


def _gate_scores_t(x, gate_w, gate_b):
    return pl.pallas_call(
        _gate_body,
        out_shape=jax.ShapeDtypeStruct((_E, _T), jnp.float32),
    )(x, gate_w, gate_b.reshape(_E, 1))


def _shared_body(x_ref, wgu_ref, wd_ref, out_ref):
    gu = lax.dot_general(x_ref[...], wgu_ref[...], (((1,), (1,)), ((), ())),
                         preferred_element_type=jnp.float32)
    g = gu[:, :_IS]
    u = gu[:, _IS:]
    h = g * jax.nn.sigmoid(g) * u
    out_ref[...] = lax.dot_general(h, wd_ref[...], (((1,), (1,)), ((), ())),
                                   preferred_element_type=jnp.float32)


def _shared_mlp(x, shared_gate_up, shared_down):
    tb = 256
    return pl.pallas_call(
        _shared_body,
        grid=(_T // tb,),
        in_specs=[
            pl.BlockSpec((tb, _H), lambda i: (i, 0)),
            pl.BlockSpec((2 * _IS, _H), lambda i: (0, 0)),
            pl.BlockSpec((_H, _IS), lambda i: (0, 0)),
        ],
        out_specs=pl.BlockSpec((tb, _H), lambda i: (i, 0)),
        out_shape=jax.ShapeDtypeStruct((_T, _H), jnp.float32),
    )(x, shared_gate_up, shared_down)


def _expert_body(be_ref, bv_ref, xs_ref, ws_ref, wgu_ref, wd_ref, y_ref):
    @pl.when(bv_ref[pl.program_id(0)] != 0)
    def _():
        gu = lax.dot_general(xs_ref[...], wgu_ref[0], (((1,), (1,)), ((), ())),
                             preferred_element_type=jnp.float32)
        g = gu[:, :_I]
        u = gu[:, _I:]
        h = g * jax.nn.sigmoid(g) * u
        y = lax.dot_general(h, wd_ref[0], (((1,), (1,)), ((), ())),
                            preferred_element_type=jnp.float32)
        y_ref[...] = y * ws_ref[...][:, :1]


def _expert_mm(x_sorted, w_sorted, block_expert, block_valid, w_gate_up, w_down):
    grid_spec = pltpu.PrefetchScalarGridSpec(
        num_scalar_prefetch=2,
        grid=(_NB,),
        in_specs=[
            pl.BlockSpec((_B, _H), lambda i, be, bv: (i, 0)),
            pl.BlockSpec((_B, 128), lambda i, be, bv: (i, 0)),
            pl.BlockSpec((1, 2 * _I, _H), lambda i, be, bv: (be[i], 0, 0)),
            pl.BlockSpec((1, _H, _I), lambda i, be, bv: (be[i], 0, 0)),
        ],
        out_specs=pl.BlockSpec((_B, _H), lambda i, be, bv: (i, 0)),
    )
    return pl.pallas_call(
        _expert_body,
        grid_spec=grid_spec,
        out_shape=jax.ShapeDtypeStruct((_P, _H), jnp.float32),
    )(block_expert, block_valid, x_sorted, w_sorted, w_gate_up, w_down)


def _top2_scan(vals):
    t1 = jnp.full((16,), _NEG, jnp.float32)
    t2 = jnp.full((16,), _NEG, jnp.float32)
    for v in vals:
        t2 = jnp.maximum(t2, jnp.minimum(t1, v))
        t1 = jnp.maximum(t1, v)
    return t1, t2


_sc_mesh = plsc.VectorSubcoreMesh(core_axis_name="c", subcore_axis_name="s")


@functools.partial(
    pl.kernel,
    out_type=(
        jax.ShapeDtypeStruct((_P, _H), jnp.float32),
        jax.ShapeDtypeStruct((_P, 128), jnp.float32),
        jax.ShapeDtypeStruct((2, _T), jnp.int32),
        jax.ShapeDtypeStruct((_NB,), jnp.int32),
        jax.ShapeDtypeStruct((_NB,), jnp.int32),
        jax.ShapeDtypeStruct((_DTILES, _E), jnp.int32),
    ),
    mesh=_sc_mesh,
    scratch_types=[
        pltpu.VMEM((_E, _DTPT), jnp.float32),
        pltpu.VMEM((_E,), jnp.float32),
        pltpu.VMEM((16, _H), jnp.float32),
        pltpu.VMEM((2 * _DNCH * 16,), jnp.int32),
        pltpu.VMEM((2 * _DNCH * 16,), jnp.int32),
        pltpu.VMEM((2 * _DNCH, 16), jnp.int32),
        pltpu.VMEM((2, _DTPT), jnp.float32),
        pltpu.VMEM((2, _DTPT), jnp.int32),
        pltpu.VMEM((16, 128), jnp.float32),
        pltpu.VMEM((_E,), jnp.int32),
        pltpu.VMEM((_DTILES, _E), jnp.int32),
        pltpu.VMEM((_NB,), jnp.int32),
        pltpu.VMEM((_NB,), jnp.int32),
        pltpu.SemaphoreType.DMA,
    ],
)
def _dispatch(scores_ref, gb_ref, x_ref,
              xs_ref, ws_ref, inv_ref, be_ref, bv_ref, cnts_ref,
              scb, gbb, xbuf, ebuf, plb, idxb, wbuf, invb, wsbuf,
              cvec, allc, beb, bvb, sem):
    sid = lax.axis_index("s")
    wid = sid
    base_tok = wid * _DTPT

    def _stage_a():
        for e in range(_E):
            pltpu.sync_copy(scores_ref.at[e, pl.ds(base_tok, _DTPT)], scb.at[e])
        pltpu.sync_copy(gb_ref, gbb)

        gbv = gbb[...]

        def chunk(c, carry):
            sraw = []
            sfc = []
            for e in range(_E):
                v = scb[e, pl.ds(c * 16, 16)]
                sraw.append(v)
                sfc.append(v + gbv[e])
            gsc = []
            for g in range(_NG):
                t1, t2 = _top2_scan(sfc[4 * g:4 * g + 4])
                gsc.append(t1 + t2)
            _, g2 = _top2_scan(gsc)
            cnt_gt = jnp.zeros((16,), jnp.int32)
            for g in range(_NG):
                cnt_gt = cnt_gt + jnp.where(gsc[g] > g2, 1, 0)
            taken = jnp.zeros((16,), jnp.int32)
            sel_g = []
            for g in range(_NG):
                sel_eq = (gsc[g] == g2) & (cnt_gt + taken < 2)
                sel_g.append((gsc[g] > g2) | sel_eq)
                taken = taken + jnp.where(sel_eq, 1, 0)
            me = [jnp.where(sel_g[e // 4], sfc[e], _NEG) for e in range(_E)]
            _, t2 = _top2_scan(me)
            cnt_gt = jnp.zeros((16,), jnp.int32)
            for e in range(_E):
                cnt_gt = cnt_gt + jnp.where(me[e] > t2, 1, 0)
            taken = jnp.zeros((16,), jnp.int32)
            nsel = jnp.zeros((16,), jnp.int32)
            id0 = jnp.zeros((16,), jnp.int32)
            id1 = jnp.zeros((16,), jnp.int32)
            w0 = jnp.zeros((16,), jnp.float32)
            w1 = jnp.zeros((16,), jnp.float32)
            for e in range(_E):
                sel_eq = (me[e] == t2) & (cnt_gt + taken < 2)
                sel = (me[e] > t2) | sel_eq
                taken = taken + jnp.where(sel_eq, 1, 0)
                take0 = sel & (nsel == 0)
                take1 = sel & (nsel == 1)
                nsel = nsel + jnp.where(sel, 1, 0)
                id0 = jnp.where(take0, e, id0)
                w0 = jnp.where(take0, sraw[e], w0)
                id1 = jnp.where(take1, e, id1)
                w1 = jnp.where(take1, sraw[e], w1)
            den = w0 + w1 + jnp.float32(1e-20)
            wbuf[0, pl.ds(c * 16, 16)] = w0 * (_SCALE / 1.0) / den
            wbuf[1, pl.ds(c * 16, 16)] = w1 * (_SCALE / 1.0) / den
            ebuf[pl.ds(32 * c, 16)] = id0
            ebuf[pl.ds(32 * c + 16, 16)] = id1
            return carry

        lax.fori_loop(0, _DNCH, chunk, 0)

        def pairvec(j, cnt):
            v = ebuf[pl.ds(16 * j, 16)]
            prior = _take16(cnt, v)
            wr = jnp.zeros((16,), jnp.int32)
            newcnt = cnt
            for e in range(_E):
                m = v == e
                cs = _cumsum16(jnp.where(m, 1, 0))
                tot = _take16(cs, jnp.full((16,), 15, jnp.int32))
                wr = jnp.where(m, cs - 1, wr)
                newcnt = newcnt + jnp.where(_iota16() == e, tot, 0)
            plb[pl.ds(16 * j, 16)] = prior + wr
            return newcnt

        cnt = lax.fori_loop(0, 2 * _DNCH, pairvec,
                            jnp.zeros((16,), jnp.int32))
        cvec[...] = cnt
        pltpu.sync_copy(cvec, cnts_ref.at[wid])

    _stage_a()
    plsc.subcore_barrier()

    def _stage_b():
        pltpu.sync_copy(cnts_ref, allc)

        def acc(w2, carry):
            tot, pri = carry
            cw = allc[w2, :]
            f = jnp.where(w2 < wid, 1, 0)
            return tot + cw, pri + cw * f

        tot, pri = lax.fori_loop(0, _DTILES, acc,
                                 (jnp.zeros((16,), jnp.int32),
                                  jnp.zeros((16,), jnp.int32)))
        padded = lax.shift_left(
            lax.shift_right_logical(tot + (_B - 1), 7), 7)
        cps = _cumsum16(padded)
        starts = cps - padded
        base = starts + pri

        for j in range(2 * _DNCH):
            c, k = j // 2, j % 2
            v = ebuf[pl.ds(16 * j, 16)]
            pos = _take16(base, v) + plb[pl.ds(16 * j, 16)]
            idxb[j, :] = pos
            invb[k, pl.ds((j // 2) * 16, 16)] = pos

        for k in range(2):
            pltpu.sync_copy(invb.at[k], inv_ref.at[k, pl.ds(base_tok, _DTPT)])

        for c in range(_DNCH):
            pltpu.sync_copy(x_ref.at[pl.ds(base_tok + c * 16, 16), :], xbuf)
            for k in range(2):
                j = 2 * c + k
                wvec = wbuf[k, pl.ds(c * 16, 16)]
                for l in range(16):
                    wsbuf[l, pl.ds(0, 16)] = jnp.full((16,), 1.0, jnp.float32) * wvec[l]
                pltpu.async_copy(wsbuf, ws_ref.at[idxb.at[j, :]], sem).wait()
                pltpu.async_copy(xbuf, xs_ref.at[idxb.at[j, :]], sem).wait()

        @pl.when(wid == 0)
        def _meta():
            nbu = lax.shift_right_logical(
                _take16(cps, jnp.full((16,), 15, jnp.int32)), 7)
            for q in range(_NB // 16):
                jv = _iota16() + 16 * q
                jb = lax.shift_left(jv, 7)
                acc2 = jnp.zeros((16,), jnp.int32)
                for e in range(_E):
                    se = _take16(starts, jnp.full((16,), e, jnp.int32))
                    acc2 = acc2 + jnp.where(jb >= se, 1, 0)
                beb[pl.ds(16 * q, 16)] = jnp.clip(acc2 - 1, 0, _E - 1)
                bvb[pl.ds(16 * q, 16)] = jnp.where(jv < nbu, 1, 0)
            pltpu.sync_copy(beb, be_ref)
            pltpu.sync_copy(bvb, bv_ref)

    _stage_b()


@functools.partial(
    pl.kernel,
    out_type=jax.ShapeDtypeStruct((_T, _H), jnp.float32),
    mesh=_sc_mesh,
    scratch_types=[
        pltpu.VMEM((2, _CTPT), jnp.int32),
        pltpu.VMEM((_CTPT, _H), jnp.float32),
        pltpu.VMEM((16, _H), jnp.float32),
        pltpu.VMEM((16, _H), jnp.float32),
        pltpu.SemaphoreType.DMA,
    ],
)
def _combine(y_ref, inv_ref, sh_ref, out_ref, ib, ob, yb0, yb1, sem):
    cid = lax.axis_index("c")
    sid = lax.axis_index("s")
    wid = sid * 2 + cid
    base = wid * _CTPT
    for k in range(2):
        pltpu.sync_copy(inv_ref.at[k, pl.ds(base, _CTPT)], ib.at[k])
    pltpu.sync_copy(sh_ref.at[pl.ds(base, _CTPT), :], ob)
    for c in range(_CTPT // 16):
        d0 = pltpu.async_copy(y_ref.at[ib.at[0, pl.ds(c * 16, 16)]], yb0, sem)
        d1 = pltpu.async_copy(y_ref.at[ib.at[1, pl.ds(c * 16, 16)]], yb1, sem)
        d0.wait()
        d1.wait()
        for l in range(16):
            r = c * 16 + l

            def qbody(q, carry):
                sl_ = pl.ds(q * 16, 16)
                ob[r, sl_] = ob[r, sl_] + yb0[l, sl_] + yb1[l, sl_]
                return carry

            lax.fori_loop(0, _H // 16, qbody, 0, unroll=4)
    pltpu.sync_copy(ob, out_ref.at[pl.ds(base, _CTPT), :])


def kernel(hidden_states, gate_w, gate_b, w_gate_up, w_down,
           shared_gate_up, shared_down):
    x = hidden_states
    scores_t = _gate_scores_t(x, gate_w, gate_b)
    xs, ws, inv, be, bv, _ = _dispatch(scores_t, gate_b, x)
    shared_out = _shared_mlp(x, shared_gate_up, shared_down)
    y = _expert_mm(xs, ws, be, bv, w_gate_up, w_down)
    return _combine(y, inv, shared_out)


# speedup vs baseline: 2.1816x; 1.1603x over previous
import functools

import jax
import jax.numpy as jnp
from jax import lax
from jax.experimental import pallas as pl
from jax.experimental.pallas import tpu as pltpu
from jax.experimental.pallas import tpu_sc as plsc

_T = 2048
_H = 1024
_E = 16
_K = 2
_NG = 4
_I = 512
_IS = 1024
_SCALE = 2.5

_B = 128
_NB = (_K * _T) // _B + _E
_P = _NB * _B

_NEG = -1e30

_DTILES = 16
_DTPT = _T // _DTILES
_DNCH = _DTPT // 16
_CTILES = 32
_CTPT = _T // _CTILES


def _iota16():
    return lax.broadcasted_iota(jnp.int32, (16,), 0)


def _take16(vec, idx):
    return lax.gather(
        vec, idx[:, None],
        lax.GatherDimensionNumbers(offset_dims=(), collapsed_slice_dims=(0,),
                                   start_index_map=(0,)),
        (1,), mode=lax.GatherScatterMode.PROMISE_IN_BOUNDS)


def _cumsum16(x):
    it = _iota16()
    for s in (1, 2, 4, 8):
        y = _take16(x, jnp.maximum(it - s, 0))
        x = x + jnp.where(it >= s, y, 0)
    return x


def _sum16v(x):
    return _take16(_cumsum16(x), jnp.full((16,), 15, jnp.int32))


def _gate_body(x_ref, w_ref, b_ref, out_ref):
    logits = lax.dot_general(w_ref[...], x_ref[...], (((1,), (1,)), ((), ())),
                             preferred_element_type=jnp.float32)
    out_ref[...] = jax.nn.sigmoid(logits + b_ref[...])


def _gate_scores_t(x, gate_w, gate_b):
    return pl.pallas_call(
        _gate_body,
        out_shape=jax.ShapeDtypeStruct((_E, _T), jnp.float32),
    )(x, gate_w, gate_b.reshape(_E, 1))


def _shared_body(x_ref, wgu_ref, wd_ref, out_ref):
    gu = lax.dot_general(x_ref[...], wgu_ref[...], (((1,), (1,)), ((), ())),
                         preferred_element_type=jnp.float32)
    g = gu[:, :_IS]
    u = gu[:, _IS:]
    h = g * jax.nn.sigmoid(g) * u
    out_ref[...] = lax.dot_general(h, wd_ref[...], (((1,), (1,)), ((), ())),
                                   preferred_element_type=jnp.float32)


def _shared_mlp(x, shared_gate_up, shared_down):
    tb = 256
    return pl.pallas_call(
        _shared_body,
        grid=(_T // tb,),
        in_specs=[
            pl.BlockSpec((tb, _H), lambda i: (i, 0)),
            pl.BlockSpec((2 * _IS, _H), lambda i: (0, 0)),
            pl.BlockSpec((_H, _IS), lambda i: (0, 0)),
        ],
        out_specs=pl.BlockSpec((tb, _H), lambda i: (i, 0)),
        out_shape=jax.ShapeDtypeStruct((_T, _H), jnp.float32),
    )(x, shared_gate_up, shared_down)


def _expert_body(be_ref, bv_ref, xs_ref, ws_ref, wgu_ref, wd_ref, y_ref):
    @pl.when(bv_ref[pl.program_id(0)] != 0)
    def _():
        gu = lax.dot_general(xs_ref[...], wgu_ref[0], (((1,), (1,)), ((), ())),
                             preferred_element_type=jnp.float32)
        g = gu[:, :_I]
        u = gu[:, _I:]
        h = g * jax.nn.sigmoid(g) * u
        y = lax.dot_general(h, wd_ref[0], (((1,), (1,)), ((), ())),
                            preferred_element_type=jnp.float32)
        y_ref[...] = y * ws_ref[...][:, :1]


def _expert_mm(x_sorted, w_sorted, block_expert, block_valid, w_gate_up, w_down):
    grid_spec = pltpu.PrefetchScalarGridSpec(
        num_scalar_prefetch=2,
        grid=(_NB,),
        in_specs=[
            pl.BlockSpec((_B, _H), lambda i, be, bv: (i, 0)),
            pl.BlockSpec((_B, 128), lambda i, be, bv: (i, 0)),
            pl.BlockSpec((1, 2 * _I, _H), lambda i, be, bv: (be[i], 0, 0)),
            pl.BlockSpec((1, _H, _I), lambda i, be, bv: (be[i], 0, 0)),
        ],
        out_specs=pl.BlockSpec((_B, _H), lambda i, be, bv: (i, 0)),
    )
    return pl.pallas_call(
        _expert_body,
        grid_spec=grid_spec,
        out_shape=jax.ShapeDtypeStruct((_P, _H), jnp.float32),
    )(block_expert, block_valid, x_sorted, w_sorted, w_gate_up, w_down)


def _top2_scan(vals):
    t1 = jnp.full((16,), _NEG, jnp.float32)
    t2 = jnp.full((16,), _NEG, jnp.float32)
    for v in vals:
        t2 = jnp.maximum(t2, jnp.minimum(t1, v))
        t1 = jnp.maximum(t1, v)
    return t1, t2


_sc_mesh = plsc.VectorSubcoreMesh(core_axis_name="c", subcore_axis_name="s")


@functools.partial(
    pl.kernel,
    out_type=(
        jax.ShapeDtypeStruct((_P, _H), jnp.float32),
        jax.ShapeDtypeStruct((_P, 128), jnp.float32),
        jax.ShapeDtypeStruct((2, _T), jnp.int32),
        jax.ShapeDtypeStruct((_NB,), jnp.int32),
        jax.ShapeDtypeStruct((_NB,), jnp.int32),
        jax.ShapeDtypeStruct((_DTILES, _E), jnp.int32),
    ),
    mesh=_sc_mesh,
    scratch_types=[
        pltpu.VMEM((_E, _DTPT), jnp.float32),
        pltpu.VMEM((_E,), jnp.float32),
        pltpu.VMEM((2, 16, _H), jnp.float32),
        pltpu.VMEM((2 * _DNCH * 16,), jnp.int32),
        pltpu.VMEM((2 * _DNCH * 16,), jnp.int32),
        pltpu.VMEM((2 * _DNCH, 16), jnp.int32),
        pltpu.VMEM((2, _DTPT), jnp.float32),
        pltpu.VMEM((2, _DTPT), jnp.int32),
        pltpu.VMEM((2, 2, 16, 128), jnp.float32),
        pltpu.VMEM((_E,), jnp.int32),
        pltpu.VMEM((_DTILES, _E), jnp.int32),
        pltpu.VMEM((_NB,), jnp.int32),
        pltpu.VMEM((_NB,), jnp.int32),
        pltpu.SemaphoreType.DMA,
    ],
)
def _dispatch(scores_ref, gb_ref, x_ref,
              xs_ref, ws_ref, inv_ref, be_ref, bv_ref, cnts_ref,
              scb, gbb, xbuf, ebuf, plb, idxb, wbuf, invb, wsbuf,
              cvec, allc, beb, bvb, sem):
    sid = lax.axis_index("s")
    wid = sid
    base_tok = wid * _DTPT

    def _stage_a():
        descs = [pltpu.async_copy(
            scores_ref.at[e, pl.ds(base_tok, _DTPT)], scb.at[e], sem)
            for e in range(_E)]
        descs.append(pltpu.async_copy(gb_ref, gbb, sem))
        for d in descs:
            d.wait()

        gbv = gbb[...]

        def chunk(c, carry):
            sraw = []
            sfc = []
            for e in range(_E):
                v = scb[e, pl.ds(c * 16, 16)]
                sraw.append(v)
                sfc.append(v + gbv[e])
            gsc = []
            for g in range(_NG):
                t1, t2 = _top2_scan(sfc[4 * g:4 * g + 4])
                gsc.append(t1 + t2)
            _, g2 = _top2_scan(gsc)
            cnt_gt = jnp.zeros((16,), jnp.int32)
            for g in range(_NG):
                cnt_gt = cnt_gt + jnp.where(gsc[g] > g2, 1, 0)
            taken = jnp.zeros((16,), jnp.int32)
            sel_g = []
            for g in range(_NG):
                sel_eq = (gsc[g] == g2) & (cnt_gt + taken < 2)
                sel_g.append((gsc[g] > g2) | sel_eq)
                taken = taken + jnp.where(sel_eq, 1, 0)
            me = [jnp.where(sel_g[e // 4], sfc[e], _NEG) for e in range(_E)]
            _, t2 = _top2_scan(me)
            cnt_gt = jnp.zeros((16,), jnp.int32)
            for e in range(_E):
                cnt_gt = cnt_gt + jnp.where(me[e] > t2, 1, 0)
            taken = jnp.zeros((16,), jnp.int32)
            nsel = jnp.zeros((16,), jnp.int32)
            id0 = jnp.zeros((16,), jnp.int32)
            id1 = jnp.zeros((16,), jnp.int32)
            w0 = jnp.zeros((16,), jnp.float32)
            w1 = jnp.zeros((16,), jnp.float32)
            for e in range(_E):
                sel_eq = (me[e] == t2) & (cnt_gt + taken < 2)
                sel = (me[e] > t2) | sel_eq
                taken = taken + jnp.where(sel_eq, 1, 0)
                take0 = sel & (nsel == 0)
                take1 = sel & (nsel == 1)
                nsel = nsel + jnp.where(sel, 1, 0)
                id0 = jnp.where(take0, e, id0)
                w0 = jnp.where(take0, sraw[e], w0)
                id1 = jnp.where(take1, e, id1)
                w1 = jnp.where(take1, sraw[e], w1)
            den = w0 + w1 + jnp.float32(1e-20)
            wbuf[0, pl.ds(c * 16, 16)] = w0 * (_SCALE / 1.0) / den
            wbuf[1, pl.ds(c * 16, 16)] = w1 * (_SCALE / 1.0) / den
            ebuf[pl.ds(32 * c, 16)] = id0
            ebuf[pl.ds(32 * c + 16, 16)] = id1
            return carry

        lax.fori_loop(0, _DNCH, chunk, 0)

        def pairvec(j, cnt):
            v = ebuf[pl.ds(16 * j, 16)]
            prior = _take16(cnt, v)
            wr = jnp.zeros((16,), jnp.int32)
            newcnt = cnt
            for e in range(_E):
                m = v == e
                cs = _cumsum16(jnp.where(m, 1, 0))
                tot = _take16(cs, jnp.full((16,), 15, jnp.int32))
                wr = jnp.where(m, cs - 1, wr)
                newcnt = newcnt + jnp.where(_iota16() == e, tot, 0)
            plb[pl.ds(16 * j, 16)] = prior + wr
            return newcnt

        cnt = lax.fori_loop(0, 2 * _DNCH, pairvec,
                            jnp.zeros((16,), jnp.int32))
        cvec[...] = cnt
        pltpu.sync_copy(cvec, cnts_ref.at[wid])

    _stage_a()
    plsc.subcore_barrier()

    def _stage_b():
        pltpu.sync_copy(cnts_ref, allc)

        def acc(w2, carry):
            tot, pri = carry
            cw = allc[w2, :]
            f = jnp.where(w2 < wid, 1, 0)
            return tot + cw, pri + cw * f

        tot, pri = lax.fori_loop(0, _DTILES, acc,
                                 (jnp.zeros((16,), jnp.int32),
                                  jnp.zeros((16,), jnp.int32)))
        padded = lax.shift_left(
            lax.shift_right_logical(tot + (_B - 1), 7), 7)
        cps = _cumsum16(padded)
        starts = cps - padded
        base = starts + pri

        for j in range(2 * _DNCH):
            c, k = j // 2, j % 2
            v = ebuf[pl.ds(16 * j, 16)]
            pos = _take16(base, v) + plb[pl.ds(16 * j, 16)]
            idxb[j, :] = pos
            invb[k, pl.ds((j // 2) * 16, 16)] = pos

        inv_descs = [pltpu.async_copy(
            invb.at[k], inv_ref.at[k, pl.ds(base_tok, _DTPT)], sem)
            for k in range(2)]

        d_pref = pltpu.async_copy(
            x_ref.at[pl.ds(base_tok, 16), :], xbuf.at[0], sem)
        sdescs = [[], []]
        for c in range(_DNCH):
            p = c % 2
            d_pref.wait()
            for k in range(2):
                wvec = wbuf[k, pl.ds(c * 16, 16)]
                for l in range(16):
                    wsbuf[p, k, l, pl.ds(0, 16)] = (
                        jnp.full((16,), 1.0, jnp.float32) * wvec[l])
            if c + 1 < _DNCH:
                pn = (c + 1) % 2
                for d in sdescs[pn]:
                    d.wait()
                sdescs[pn] = []
                d_pref = pltpu.async_copy(
                    x_ref.at[pl.ds(base_tok + (c + 1) * 16, 16), :],
                    xbuf.at[pn], sem)
            for k in range(2):
                j = 2 * c + k
                sdescs[p].append(pltpu.async_copy(
                    wsbuf.at[p, k], ws_ref.at[idxb.at[j, :]], sem))
                sdescs[p].append(pltpu.async_copy(
                    xbuf.at[p], xs_ref.at[idxb.at[j, :]], sem))
        for pp in (0, 1):
            for d in sdescs[pp]:
                d.wait()
        for d in inv_descs:
            d.wait()

        @pl.when(wid == 0)
        def _meta():
            nbu = lax.shift_right_logical(
                _take16(cps, jnp.full((16,), 15, jnp.int32)), 7)
            for q in range(_NB // 16):
                jv = _iota16() + 16 * q
                jb = lax.shift_left(jv, 7)
                acc2 = jnp.zeros((16,), jnp.int32)
                for e in range(_E):
                    se = _take16(starts, jnp.full((16,), e, jnp.int32))
                    acc2 = acc2 + jnp.where(jb >= se, 1, 0)
                beb[pl.ds(16 * q, 16)] = jnp.clip(acc2 - 1, 0, _E - 1)
                bvb[pl.ds(16 * q, 16)] = jnp.where(jv < nbu, 1, 0)
            pltpu.sync_copy(beb, be_ref)
            pltpu.sync_copy(bvb, bv_ref)

    _stage_b()


@functools.partial(
    pl.kernel,
    out_type=jax.ShapeDtypeStruct((_T, _H), jnp.float32),
    mesh=_sc_mesh,
    scratch_types=[
        pltpu.VMEM((2, _CTPT), jnp.int32),
        pltpu.VMEM((2, 16, _H), jnp.float32),
        pltpu.VMEM((2, 16, _H), jnp.float32),
        pltpu.VMEM((2, 16, _H), jnp.float32),
        pltpu.SemaphoreType.DMA,
    ],
)
def _combine(y_ref, inv_ref, sh_ref, out_ref, ib, sob, yb0, yb1, sem):
    cid = lax.axis_index("c")
    sid = lax.axis_index("s")
    wid = sid * 2 + cid
    base = wid * _CTPT
    nch = _CTPT // 16
    for k in range(2):
        pltpu.sync_copy(inv_ref.at[k, pl.ds(base, _CTPT)], ib.at[k])

    def _issue(c, p):
        return [
            pltpu.async_copy(sh_ref.at[pl.ds(base + c * 16, 16), :],
                             sob.at[p], sem),
            pltpu.async_copy(y_ref.at[ib.at[0, pl.ds(c * 16, 16)]],
                             yb0.at[p], sem),
            pltpu.async_copy(y_ref.at[ib.at[1, pl.ds(c * 16, 16)]],
                             yb1.at[p], sem),
        ]

    dg = _issue(0, 0)
    dw = [None, None]
    for c in range(nch):
        p = c % 2
        for d in dg:
            d.wait()
        if c + 1 < nch:
            pn = (c + 1) % 2
            if dw[pn] is not None:
                dw[pn].wait()
                dw[pn] = None
            dg = _issue(c + 1, pn)
        for l in range(16):

            def qbody(q, carry):
                sl_ = pl.ds(q * 16, 16)
                sob[p, l, sl_] = sob[p, l, sl_] + yb0[p, l, sl_] + yb1[p, l, sl_]
                return carry

            lax.fori_loop(0, _H // 16, qbody, 0, unroll=4)
        dw[p] = pltpu.async_copy(
            sob.at[p], out_ref.at[pl.ds(base + c * 16, 16), :], sem)
    for d in dw:
        if d is not None:
            d.wait()


def kernel(hidden_states, gate_w, gate_b, w_gate_up, w_down,
           shared_gate_up, shared_down):
    x = hidden_states
    scores_t = _gate_scores_t(x, gate_w, gate_b)
    xs, ws, inv, be, bv, _ = _dispatch(scores_t, gate_b, x)
    shared_out = _shared_mlp(x, shared_gate_up, shared_down)
    y = _expert_mm(xs, ws, be, bv, w_gate_up, w_down)
    return _combine(y, inv, shared_out)


# speedup vs baseline: 2.5379x; 1.1633x over previous
import functools

import jax
import jax.numpy as jnp
from jax import lax
from jax.experimental import pallas as pl
from jax.experimental.pallas import tpu as pltpu
from jax.experimental.pallas import tpu_sc as plsc

_T = 2048
_H = 1024
_E = 16
_K = 2
_NG = 4
_I = 512
_IS = 1024
_SCALE = 2.5

_B = 256
_BSH = 8
_NB = (_K * _T) // _B + _E
_P = _NB * _B

_NEG = -1e30

_DTILES = 16
_DTPT = _T // _DTILES
_DNCH = _DTPT // 16
_CTILES = 32
_CTPT = _T // _CTILES


def _iota16():
    return lax.broadcasted_iota(jnp.int32, (16,), 0)


def _take16(vec, idx):
    return lax.gather(
        vec, idx[:, None],
        lax.GatherDimensionNumbers(offset_dims=(), collapsed_slice_dims=(0,),
                                   start_index_map=(0,)),
        (1,), mode=lax.GatherScatterMode.PROMISE_IN_BOUNDS)


def _cumsum16(x):
    it = _iota16()
    for s in (1, 2, 4, 8):
        y = _take16(x, jnp.maximum(it - s, 0))
        x = x + jnp.where(it >= s, y, 0)
    return x


def _sum16v(x):
    return _take16(_cumsum16(x), jnp.full((16,), 15, jnp.int32))


def _gate_body(x_ref, w_ref, b_ref, out_ref):
    logits = lax.dot_general(w_ref[...], x_ref[...], (((1,), (1,)), ((), ())),
                             preferred_element_type=jnp.float32)
    out_ref[...] = jax.nn.sigmoid(logits + b_ref[...])


def _gate_scores_t(x, gate_w, gate_b):
    return pl.pallas_call(
        _gate_body,
        out_shape=jax.ShapeDtypeStruct((_E, _T), jnp.float32),
    )(x, gate_w, gate_b.reshape(_E, 1))


def _shared_body(x_ref, wgu_ref, wd_ref, out_ref):
    gu = lax.dot_general(x_ref[...], wgu_ref[...], (((1,), (1,)), ((), ())),
                         preferred_element_type=jnp.float32)
    g = gu[:, :_IS]
    u = gu[:, _IS:]
    h = g * jax.nn.sigmoid(g) * u
    out_ref[...] = lax.dot_general(h, wd_ref[...], (((1,), (1,)), ((), ())),
                                   preferred_element_type=jnp.float32)


def _shared_mlp(x, shared_gate_up, shared_down):
    tb = 256
    return pl.pallas_call(
        _shared_body,
        grid=(_T // tb,),
        in_specs=[
            pl.BlockSpec((tb, _H), lambda i: (i, 0)),
            pl.BlockSpec((2 * _IS, _H), lambda i: (0, 0)),
            pl.BlockSpec((_H, _IS), lambda i: (0, 0)),
        ],
        out_specs=pl.BlockSpec((tb, _H), lambda i: (i, 0)),
        out_shape=jax.ShapeDtypeStruct((_T, _H), jnp.float32),
    )(x, shared_gate_up, shared_down)


def _expert_body(be_ref, bv_ref, xs_ref, ws_ref, wgu_ref, wd_ref, y_ref):
    @pl.when(bv_ref[pl.program_id(0)] != 0)
    def _():
        gu = lax.dot_general(xs_ref[...], wgu_ref[0], (((1,), (1,)), ((), ())),
                             preferred_element_type=jnp.float32)
        g = gu[:, :_I]
        u = gu[:, _I:]
        h = g * jax.nn.sigmoid(g) * u
        y = lax.dot_general(h, wd_ref[0], (((1,), (1,)), ((), ())),
                            preferred_element_type=jnp.float32)
        y_ref[...] = y * ws_ref[...][:, :1]


def _expert_mm(x_sorted, w_sorted, block_expert, block_valid, w_gate_up, w_down):
    grid_spec = pltpu.PrefetchScalarGridSpec(
        num_scalar_prefetch=2,
        grid=(_NB,),
        in_specs=[
            pl.BlockSpec((_B, _H), lambda i, be, bv: (i, 0)),
            pl.BlockSpec((_B, 128), lambda i, be, bv: (i, 0)),
            pl.BlockSpec((1, 2 * _I, _H), lambda i, be, bv: (be[i], 0, 0)),
            pl.BlockSpec((1, _H, _I), lambda i, be, bv: (be[i], 0, 0)),
        ],
        out_specs=pl.BlockSpec((_B, _H), lambda i, be, bv: (i, 0)),
    )
    return pl.pallas_call(
        _expert_body,
        grid_spec=grid_spec,
        out_shape=jax.ShapeDtypeStruct((_P, _H), jnp.float32),
    )(block_expert, block_valid, x_sorted, w_sorted, w_gate_up, w_down)


def _top2_scan(vals):
    t1 = jnp.full((16,), _NEG, jnp.float32)
    t2 = jnp.full((16,), _NEG, jnp.float32)
    for v in vals:
        t2 = jnp.maximum(t2, jnp.minimum(t1, v))
        t1 = jnp.maximum(t1, v)
    return t1, t2


_sc_mesh = plsc.VectorSubcoreMesh(core_axis_name="c", subcore_axis_name="s")


@functools.partial(
    pl.kernel,
    out_type=(
        jax.ShapeDtypeStruct((_P, _H), jnp.float32),
        jax.ShapeDtypeStruct((_P, 128), jnp.float32),
        jax.ShapeDtypeStruct((2, _T), jnp.int32),
        jax.ShapeDtypeStruct((_NB,), jnp.int32),
        jax.ShapeDtypeStruct((_NB,), jnp.int32),
        jax.ShapeDtypeStruct((_DTILES, _E), jnp.int32),
    ),
    mesh=_sc_mesh,
    scratch_types=[
        pltpu.VMEM((_E, _DTPT), jnp.float32),
        pltpu.VMEM((_E,), jnp.float32),
        pltpu.VMEM((2, 16, _H), jnp.float32),
        pltpu.VMEM((2 * _DNCH * 16,), jnp.int32),
        pltpu.VMEM((2 * _DNCH * 16,), jnp.int32),
        pltpu.VMEM((2 * _DNCH, 16), jnp.int32),
        pltpu.VMEM((2, _DTPT), jnp.float32),
        pltpu.VMEM((2, _DTPT), jnp.int32),
        pltpu.VMEM((2, 2, 16, 128), jnp.float32),
        pltpu.VMEM((_E,), jnp.int32),
        pltpu.VMEM((_DTILES, _E), jnp.int32),
        pltpu.VMEM((_NB,), jnp.int32),
        pltpu.VMEM((_NB,), jnp.int32),
        pltpu.SemaphoreType.DMA,
    ],
)
def _dispatch(scores_ref, gb_ref, x_ref,
              xs_ref, ws_ref, inv_ref, be_ref, bv_ref, cnts_ref,
              scb, gbb, xbuf, ebuf, plb, idxb, wbuf, invb, wsbuf,
              cvec, allc, beb, bvb, sem):
    sid = lax.axis_index("s")
    wid = sid
    base_tok = wid * _DTPT

    def _stage_a():
        descs = [pltpu.async_copy(
            scores_ref.at[e, pl.ds(base_tok, _DTPT)], scb.at[e], sem)
            for e in range(_E)]
        descs.append(pltpu.async_copy(gb_ref, gbb, sem))
        for d in descs:
            d.wait()

        gbv = gbb[...]

        def chunk(c, carry):
            sraw = []
            sfc = []
            for e in range(_E):
                v = scb[e, pl.ds(c * 16, 16)]
                sraw.append(v)
                sfc.append(v + gbv[e])
            gsc = []
            for g in range(_NG):
                t1, t2 = _top2_scan(sfc[4 * g:4 * g + 4])
                gsc.append(t1 + t2)
            _, g2 = _top2_scan(gsc)
            cnt_gt = jnp.zeros((16,), jnp.int32)
            for g in range(_NG):
                cnt_gt = cnt_gt + jnp.where(gsc[g] > g2, 1, 0)
            taken = jnp.zeros((16,), jnp.int32)
            sel_g = []
            for g in range(_NG):
                sel_eq = (gsc[g] == g2) & (cnt_gt + taken < 2)
                sel_g.append((gsc[g] > g2) | sel_eq)
                taken = taken + jnp.where(sel_eq, 1, 0)
            me = [jnp.where(sel_g[e // 4], sfc[e], _NEG) for e in range(_E)]
            _, t2 = _top2_scan(me)
            cnt_gt = jnp.zeros((16,), jnp.int32)
            for e in range(_E):
                cnt_gt = cnt_gt + jnp.where(me[e] > t2, 1, 0)
            taken = jnp.zeros((16,), jnp.int32)
            nsel = jnp.zeros((16,), jnp.int32)
            id0 = jnp.zeros((16,), jnp.int32)
            id1 = jnp.zeros((16,), jnp.int32)
            w0 = jnp.zeros((16,), jnp.float32)
            w1 = jnp.zeros((16,), jnp.float32)
            for e in range(_E):
                sel_eq = (me[e] == t2) & (cnt_gt + taken < 2)
                sel = (me[e] > t2) | sel_eq
                taken = taken + jnp.where(sel_eq, 1, 0)
                take0 = sel & (nsel == 0)
                take1 = sel & (nsel == 1)
                nsel = nsel + jnp.where(sel, 1, 0)
                id0 = jnp.where(take0, e, id0)
                w0 = jnp.where(take0, sraw[e], w0)
                id1 = jnp.where(take1, e, id1)
                w1 = jnp.where(take1, sraw[e], w1)
            den = w0 + w1 + jnp.float32(1e-20)
            wbuf[0, pl.ds(c * 16, 16)] = w0 * (_SCALE / 1.0) / den
            wbuf[1, pl.ds(c * 16, 16)] = w1 * (_SCALE / 1.0) / den
            ebuf[pl.ds(32 * c, 16)] = id0
            ebuf[pl.ds(32 * c + 16, 16)] = id1
            return carry

        lax.fori_loop(0, _DNCH, chunk, 0)

        def pairvec(j, cnt):
            v = ebuf[pl.ds(16 * j, 16)]
            prior = _take16(cnt, v)
            wr = jnp.zeros((16,), jnp.int32)
            newcnt = cnt
            for e in range(_E):
                m = v == e
                cs = _cumsum16(jnp.where(m, 1, 0))
                tot = _take16(cs, jnp.full((16,), 15, jnp.int32))
                wr = jnp.where(m, cs - 1, wr)
                newcnt = newcnt + jnp.where(_iota16() == e, tot, 0)
            plb[pl.ds(16 * j, 16)] = prior + wr
            return newcnt

        cnt = lax.fori_loop(0, 2 * _DNCH, pairvec,
                            jnp.zeros((16,), jnp.int32))
        cvec[...] = cnt
        pltpu.sync_copy(cvec, cnts_ref.at[wid])

    _stage_a()
    plsc.subcore_barrier()

    def _stage_b():
        pltpu.sync_copy(cnts_ref, allc)

        def acc(w2, carry):
            tot, pri = carry
            cw = allc[w2, :]
            f = jnp.where(w2 < wid, 1, 0)
            return tot + cw, pri + cw * f

        tot, pri = lax.fori_loop(0, _DTILES, acc,
                                 (jnp.zeros((16,), jnp.int32),
                                  jnp.zeros((16,), jnp.int32)))
        padded = lax.shift_left(
            lax.shift_right_logical(tot + (_B - 1), _BSH), _BSH)
        cps = _cumsum16(padded)
        starts = cps - padded
        base = starts + pri

        for j in range(2 * _DNCH):
            c, k = j // 2, j % 2
            v = ebuf[pl.ds(16 * j, 16)]
            pos = _take16(base, v) + plb[pl.ds(16 * j, 16)]
            idxb[j, :] = pos
            invb[k, pl.ds((j // 2) * 16, 16)] = pos

        inv_descs = [pltpu.async_copy(
            invb.at[k], inv_ref.at[k, pl.ds(base_tok, _DTPT)], sem)
            for k in range(2)]

        d_pref = pltpu.async_copy(
            x_ref.at[pl.ds(base_tok, 16), :], xbuf.at[0], sem)
        sdescs = [[], []]
        for c in range(_DNCH):
            p = c % 2
            d_pref.wait()
            for k in range(2):
                wvec = wbuf[k, pl.ds(c * 16, 16)]
                for l in range(16):
                    wsbuf[p, k, l, pl.ds(0, 16)] = (
                        jnp.full((16,), 1.0, jnp.float32) * wvec[l])
            if c + 1 < _DNCH:
                pn = (c + 1) % 2
                for d in sdescs[pn]:
                    d.wait()
                sdescs[pn] = []
                d_pref = pltpu.async_copy(
                    x_ref.at[pl.ds(base_tok + (c + 1) * 16, 16), :],
                    xbuf.at[pn], sem)
            for k in range(2):
                j = 2 * c + k
                sdescs[p].append(pltpu.async_copy(
                    wsbuf.at[p, k], ws_ref.at[idxb.at[j, :]], sem))
                sdescs[p].append(pltpu.async_copy(
                    xbuf.at[p], xs_ref.at[idxb.at[j, :]], sem))
        for pp in (0, 1):
            for d in sdescs[pp]:
                d.wait()
        for d in inv_descs:
            d.wait()

        @pl.when(wid == 0)
        def _meta():
            nbu = lax.shift_right_logical(
                _take16(cps, jnp.full((16,), 15, jnp.int32)), _BSH)
            for q in range(_NB // 16):
                jv = _iota16() + 16 * q
                jb = lax.shift_left(jv, _BSH)
                acc2 = jnp.zeros((16,), jnp.int32)
                for e in range(_E):
                    se = _take16(starts, jnp.full((16,), e, jnp.int32))
                    acc2 = acc2 + jnp.where(jb >= se, 1, 0)
                beb[pl.ds(16 * q, 16)] = jnp.clip(acc2 - 1, 0, _E - 1)
                bvb[pl.ds(16 * q, 16)] = jnp.where(jv < nbu, 1, 0)
            pltpu.sync_copy(beb, be_ref)
            pltpu.sync_copy(bvb, bv_ref)

    _stage_b()


@functools.partial(
    pl.kernel,
    out_type=jax.ShapeDtypeStruct((_T, _H), jnp.float32),
    mesh=_sc_mesh,
    scratch_types=[
        pltpu.VMEM((2, _CTPT), jnp.int32),
        pltpu.VMEM((2, 16, _H), jnp.float32),
        pltpu.VMEM((2, 16, _H), jnp.float32),
        pltpu.VMEM((2, 16, _H), jnp.float32),
        pltpu.SemaphoreType.DMA,
    ],
)
def _combine(y_ref, inv_ref, sh_ref, out_ref, ib, sob, yb0, yb1, sem):
    cid = lax.axis_index("c")
    sid = lax.axis_index("s")
    wid = sid * 2 + cid
    base = wid * _CTPT
    nch = _CTPT // 16
    for k in range(2):
        pltpu.sync_copy(inv_ref.at[k, pl.ds(base, _CTPT)], ib.at[k])

    def _issue(c, p):
        return [
            pltpu.async_copy(sh_ref.at[pl.ds(base + c * 16, 16), :],
                             sob.at[p], sem),
            pltpu.async_copy(y_ref.at[ib.at[0, pl.ds(c * 16, 16)]],
                             yb0.at[p], sem),
            pltpu.async_copy(y_ref.at[ib.at[1, pl.ds(c * 16, 16)]],
                             yb1.at[p], sem),
        ]

    dg = _issue(0, 0)
    dw = [None, None]
    for c in range(nch):
        p = c % 2
        for d in dg:
            d.wait()
        if c + 1 < nch:
            pn = (c + 1) % 2
            if dw[pn] is not None:
                dw[pn].wait()
                dw[pn] = None
            dg = _issue(c + 1, pn)
        for l in range(16):

            def qbody(q, carry):
                sl_ = pl.ds(q * 16, 16)
                sob[p, l, sl_] = sob[p, l, sl_] + yb0[p, l, sl_] + yb1[p, l, sl_]
                return carry

            lax.fori_loop(0, _H // 16, qbody, 0, unroll=4)
        dw[p] = pltpu.async_copy(
            sob.at[p], out_ref.at[pl.ds(base + c * 16, 16), :], sem)
    for d in dw:
        if d is not None:
            d.wait()


def kernel(hidden_states, gate_w, gate_b, w_gate_up, w_down,
           shared_gate_up, shared_down):
    x = hidden_states
    scores_t = _gate_scores_t(x, gate_w, gate_b)
    xs, ws, inv, be, bv, _ = _dispatch(scores_t, gate_b, x)
    shared_out = _shared_mlp(x, shared_gate_up, shared_down)
    y = _expert_mm(xs, ws, be, bv, w_gate_up, w_down)
    return _combine(y, inv, shared_out)
